# baseline (device time: 16078 ns/iter reference)
import jax
import jax.numpy as jnp
from jax import lax
from jax.experimental import pallas as pl
from jax.experimental.pallas import tpu as pltpu

N_DEV = 4
B, SQ, SKV = 2, 256, 256
H_LOC, DH = 4, 64
D_MODEL = 512
D_CTX = H_LOC * DH
HALF = D_CTX // 2
BLK = 64


def kernel(x, Wq, K_ext, V_ext, Wo):
    my = lax.axis_index("i")
    Wq_loc = lax.dynamic_slice_in_dim(Wq, my * D_CTX, D_CTX, axis=1)
    x2d = x.reshape(B * SQ, D_MODEL).astype(jnp.bfloat16)
    Wq_bf = Wq_loc.astype(jnp.bfloat16)
    Wo_bf = Wo.astype(jnp.bfloat16)
    Kt = K_ext.transpose(0, 2, 1, 3).astype(jnp.bfloat16)
    Vt = V_ext.transpose(0, 2, 1, 3).astype(jnp.bfloat16)

    def body(x_ref, wq_ref, k_ref, v_ref, wo_ref, out_ref,
             own_ref, chunk_l_ref, chunk_r_ref, half_cw_ref, half_ccw_ref,
             send_sems, recv_sems):
        my_pos = lax.axis_index("i")
        left = lax.rem(my_pos - 1 + N_DEV, N_DEV)
        right = lax.rem(my_pos + 1, N_DEV)

        barrier_sem = pltpu.get_barrier_semaphore()
        for nbr in (left, right):
            pl.semaphore_signal(
                barrier_sem, inc=1,
                device_id=(nbr,), device_id_type=pl.DeviceIdType.MESH,
            )

        q2d = jnp.dot(x_ref[...], wq_ref[...],
                      preferred_element_type=jnp.float32)

        qb = lax.broadcasted_iota(jnp.int32, (SQ, SKV), 0) // BLK
        kb = lax.broadcasted_iota(jnp.int32, (SQ, SKV), 1) // BLK
        mask = kb <= qb

        def head_ctx(b, h):
            q = q2d[b * SQ:(b + 1) * SQ, h * DH:(h + 1) * DH]
            s = lax.dot_general(
                q.astype(jnp.bfloat16), k_ref[b, h], (((1,), (1,)), ((), ())),
                preferred_element_type=jnp.float32) * 0.125
            s = jnp.where(mask, s, -1e9)
            m = jnp.max(s, axis=1, keepdims=True)
            w = jnp.exp(s - m)
            w = w / jnp.sum(w, axis=1, keepdims=True)
            return jnp.dot(w.astype(jnp.bfloat16), v_ref[b, h],
                           preferred_element_type=jnp.float32)

        def compute_half(half):
            rows = []
            for b in range(B):
                rows.append(jnp.concatenate(
                    [head_ctx(b, 2 * half), head_ctx(b, 2 * half + 1)], axis=1))
            return jnp.concatenate(rows, axis=0).astype(jnp.bfloat16)

        def direct(half, sem, target, dst_ref):
            return pltpu.make_async_remote_copy(
                src_ref=own_ref.at[half], dst_ref=dst_ref.at[half],
                send_sem=send_sems.at[sem], recv_sem=recv_sems.at[sem],
                device_id=(target,), device_id_type=pl.DeviceIdType.MESH,
            )

        ctx_h0 = compute_half(0)
        own_ref[0] = ctx_h0
        pl.semaphore_wait(barrier_sem, 2)
        d_cw0 = direct(0, 0, right, chunk_l_ref)
        d_ccw0 = direct(0, 1, left, chunk_r_ref)
        d_cw0.start()
        d_ccw0.start()

        ctx_h1 = compute_half(1)
        own_ref[1] = ctx_h1
        d_cw1 = direct(1, 2, right, chunk_l_ref)
        d_ccw1 = direct(1, 3, left, chunk_r_ref)
        d_cw1.start()
        d_ccw1.start()

        def wo_dot(chunk_half, origin, h):
            return jnp.dot(
                chunk_half,
                wo_ref[pl.ds(origin * D_CTX + h * HALF, HALF), :],
                preferred_element_type=jnp.float32)

        acc = wo_dot(ctx_h0, my_pos, 0) + wo_dot(ctx_h1, my_pos, 1)

        d_cw0.wait_recv()
        relay_cw = pltpu.make_async_remote_copy(
            src_ref=chunk_l_ref.at[0], dst_ref=half_cw_ref,
            send_sem=send_sems.at[4], recv_sem=recv_sems.at[4],
            device_id=(right,), device_id_type=pl.DeviceIdType.MESH,
        )
        relay_cw.start()
        d_ccw0.wait_recv()
        acc = acc + wo_dot(chunk_l_ref[0], left, 0) + wo_dot(chunk_r_ref[0], right, 0)

        d_ccw1.wait_recv()
        relay_ccw = pltpu.make_async_remote_copy(
            src_ref=chunk_r_ref.at[1], dst_ref=half_ccw_ref,
            send_sem=send_sems.at[5], recv_sem=recv_sems.at[5],
            device_id=(left,), device_id_type=pl.DeviceIdType.MESH,
        )
        relay_ccw.start()
        d_cw1.wait_recv()
        acc = acc + wo_dot(chunk_l_ref[1], left, 1) + wo_dot(chunk_r_ref[1], right, 1)

        opp = lax.rem(my_pos + 2, N_DEV)
        relay_cw.wait_recv()
        acc = acc + wo_dot(half_cw_ref[...], opp, 0)
        relay_ccw.wait_recv()
        acc = acc + wo_dot(half_ccw_ref[...], opp, 1)

        out_ref[...] = acc

        d_cw0.wait_send()
        d_ccw0.wait_send()
        d_cw1.wait_send()
        d_ccw1.wait_send()
        relay_cw.wait_send()
        relay_ccw.wait_send()

    out2d = pl.pallas_call(
        body,
        out_shape=jax.ShapeDtypeStruct((B * SQ, D_MODEL), jnp.float32),
        in_specs=[pl.BlockSpec(memory_space=pltpu.VMEM)] * 5,
        out_specs=pl.BlockSpec(memory_space=pltpu.VMEM),
        scratch_shapes=[
            pltpu.VMEM((2, B * SQ, HALF), jnp.bfloat16),
            pltpu.VMEM((2, B * SQ, HALF), jnp.bfloat16),
            pltpu.VMEM((2, B * SQ, HALF), jnp.bfloat16),
            pltpu.VMEM((B * SQ, HALF), jnp.bfloat16),
            pltpu.VMEM((B * SQ, HALF), jnp.bfloat16),
            pltpu.SemaphoreType.DMA((6,)),
            pltpu.SemaphoreType.DMA((6,)),
        ],
        compiler_params=pltpu.CompilerParams(collective_id=0),
    )(x2d, Wq_bf, Kt, Vt, Wo_bf)
    return out2d.reshape(B, SQ, D_MODEL)


# device time: 7206 ns/iter; 2.2312x vs baseline; 2.2312x over previous
import jax
import jax.numpy as jnp
from jax import lax
from jax.experimental import pallas as pl
from jax.experimental.pallas import tpu as pltpu

N_DEV = 4
B, SQ, SKV = 2, 256, 256
H_LOC, DH = 4, 64
D_MODEL = 512
D_CTX = H_LOC * DH
HALF = D_CTX // 2
BLK = 64


def kernel(x, Wq, K_ext, V_ext, Wo):
    my = lax.axis_index("i")
    Wq_loc = lax.dynamic_slice_in_dim(Wq, my * D_CTX, D_CTX, axis=1)
    x2d = x.reshape(B * SQ, D_MODEL)
    Kt = K_ext.transpose(0, 2, 1, 3)
    Vt = V_ext.transpose(0, 2, 1, 3)

    def body(x_ref, wq_ref, k_ref, v_ref, wo_ref, out_ref,
             own_ref, chunk_l_ref, chunk_r_ref, half_cw_ref, half_ccw_ref,
             send_sems, recv_sems):
        my_pos = lax.axis_index("i")
        left = lax.rem(my_pos - 1 + N_DEV, N_DEV)
        right = lax.rem(my_pos + 1, N_DEV)

        barrier_sem = pltpu.get_barrier_semaphore()
        for nbr in (left, right):
            pl.semaphore_signal(
                barrier_sem, inc=1,
                device_id=(nbr,), device_id_type=pl.DeviceIdType.MESH,
            )

        q2d = jnp.dot(x_ref[...].astype(jnp.bfloat16),
                      wq_ref[...].astype(jnp.bfloat16),
                      preferred_element_type=jnp.float32)

        qb = lax.broadcasted_iota(jnp.int32, (SQ, SKV), 0) // BLK
        kb = lax.broadcasted_iota(jnp.int32, (SQ, SKV), 1) // BLK
        mask = kb <= qb

        def head_ctx(b, h):
            q = q2d[b * SQ:(b + 1) * SQ, h * DH:(h + 1) * DH]
            s = lax.dot_general(
                q.astype(jnp.bfloat16), k_ref[b, h].astype(jnp.bfloat16),
                (((1,), (1,)), ((), ())),
                preferred_element_type=jnp.float32) * 0.125
            s = jnp.where(mask, s, -1e9)
            m = jnp.max(s, axis=1, keepdims=True)
            w = jnp.exp(s - m)
            w = w / jnp.sum(w, axis=1, keepdims=True)
            return jnp.dot(w.astype(jnp.bfloat16),
                           v_ref[b, h].astype(jnp.bfloat16),
                           preferred_element_type=jnp.float32)

        def compute_half(half):
            rows = []
            for b in range(B):
                rows.append(jnp.concatenate(
                    [head_ctx(b, 2 * half), head_ctx(b, 2 * half + 1)], axis=1))
            return jnp.concatenate(rows, axis=0).astype(jnp.bfloat16)

        def direct(half, sem, target, dst_ref):
            return pltpu.make_async_remote_copy(
                src_ref=own_ref.at[half], dst_ref=dst_ref.at[half],
                send_sem=send_sems.at[sem], recv_sem=recv_sems.at[sem],
                device_id=(target,), device_id_type=pl.DeviceIdType.MESH,
            )

        ctx_h0 = compute_half(0)
        own_ref[0] = ctx_h0
        pl.semaphore_wait(barrier_sem, 2)
        d_cw0 = direct(0, 0, right, chunk_l_ref)
        d_ccw0 = direct(0, 1, left, chunk_r_ref)
        d_cw0.start()
        d_ccw0.start()

        ctx_h1 = compute_half(1)
        own_ref[1] = ctx_h1
        d_cw1 = direct(1, 2, right, chunk_l_ref)
        d_ccw1 = direct(1, 3, left, chunk_r_ref)
        d_cw1.start()
        d_ccw1.start()

        def wo_dot(chunk_half, origin, h):
            return jnp.dot(
                chunk_half,
                wo_ref[pl.ds(origin * D_CTX + h * HALF, HALF), :].astype(
                    jnp.bfloat16),
                preferred_element_type=jnp.float32)

        acc = wo_dot(ctx_h0, my_pos, 0) + wo_dot(ctx_h1, my_pos, 1)

        d_cw0.wait_recv()
        relay_cw = pltpu.make_async_remote_copy(
            src_ref=chunk_l_ref.at[0], dst_ref=half_cw_ref,
            send_sem=send_sems.at[4], recv_sem=recv_sems.at[4],
            device_id=(right,), device_id_type=pl.DeviceIdType.MESH,
        )
        relay_cw.start()
        d_ccw0.wait_recv()
        acc = acc + wo_dot(chunk_l_ref[0], left, 0) + wo_dot(chunk_r_ref[0], right, 0)

        d_ccw1.wait_recv()
        relay_ccw = pltpu.make_async_remote_copy(
            src_ref=chunk_r_ref.at[1], dst_ref=half_ccw_ref,
            send_sem=send_sems.at[5], recv_sem=recv_sems.at[5],
            device_id=(left,), device_id_type=pl.DeviceIdType.MESH,
        )
        relay_ccw.start()
        d_cw1.wait_recv()
        acc = acc + wo_dot(chunk_l_ref[1], left, 1) + wo_dot(chunk_r_ref[1], right, 1)

        opp = lax.rem(my_pos + 2, N_DEV)
        relay_cw.wait_recv()
        acc = acc + wo_dot(half_cw_ref[...], opp, 0)
        relay_ccw.wait_recv()
        acc = acc + wo_dot(half_ccw_ref[...], opp, 1)

        out_ref[...] = acc

        d_cw0.wait_send()
        d_ccw0.wait_send()
        d_cw1.wait_send()
        d_ccw1.wait_send()
        relay_cw.wait_send()
        relay_ccw.wait_send()

    out2d = pl.pallas_call(
        body,
        out_shape=jax.ShapeDtypeStruct((B * SQ, D_MODEL), jnp.float32),
        in_specs=[pl.BlockSpec(memory_space=pltpu.VMEM)] * 5,
        out_specs=pl.BlockSpec(memory_space=pltpu.VMEM),
        scratch_shapes=[
            pltpu.VMEM((2, B * SQ, HALF), jnp.bfloat16),
            pltpu.VMEM((2, B * SQ, HALF), jnp.bfloat16),
            pltpu.VMEM((2, B * SQ, HALF), jnp.bfloat16),
            pltpu.VMEM((B * SQ, HALF), jnp.bfloat16),
            pltpu.VMEM((B * SQ, HALF), jnp.bfloat16),
            pltpu.SemaphoreType.DMA((6,)),
            pltpu.SemaphoreType.DMA((6,)),
        ],
        compiler_params=pltpu.CompilerParams(collective_id=0),
    )(x2d, Wq_loc, Kt, Vt, Wo)
    return out2d.reshape(B, SQ, D_MODEL)
